# baseline (speedup 1.0000x reference)
import functools

import jax
import jax.numpy as jnp
from jax import lax
from jax.experimental import pallas as pl
from jax.experimental.pallas import tpu as pltpu
from jax.experimental.pallas import tpu_sc as plsc


def _make_pool(V, D, B, H):
    info = plsc.get_sparse_core_info()
    NC, NS = info.num_cores, info.num_subcores
    NW = NC * NS
    assert B % NW == 0 and D == 64
    BPW = B // NW
    mesh = plsc.VectorSubcoreMesh(core_axis_name="c", subcore_axis_name="s")

    @functools.partial(
        pl.kernel,
        mesh=mesh,
        compiler_params=pltpu.CompilerParams(use_tc_tiling_on_sc=False),
        out_type=jax.ShapeDtypeStruct((B, D), jnp.float32),
        scratch_types=[
            pltpu.VMEM((BPW, H), jnp.int32),
            pltpu.VMEM((H, D), jnp.float32),
            pltpu.VMEM((BPW, D), jnp.float32),
            pltpu.SemaphoreType.DMA,
        ],
    )
    def pool(ctx_hbm, table_hbm, out_hbm, idx_v, rows_v, acc_v, sem):
        wid = lax.axis_index("s") * NC + lax.axis_index("c")
        base = wid * BPW
        pltpu.sync_copy(ctx_hbm.at[pl.ds(base, BPW)], idx_v)
        scale = jnp.float32(1.0 / H)

        def body(j, carry):
            pltpu.async_copy(table_hbm.at[idx_v.at[j]], rows_v, sem).wait()

            def hbody(h, acc):
                a0, a1, a2, a3 = acc
                return (
                    a0 + rows_v[h, pl.ds(0, 16)],
                    a1 + rows_v[h, pl.ds(16, 16)],
                    a2 + rows_v[h, pl.ds(32, 16)],
                    a3 + rows_v[h, pl.ds(48, 16)],
                )

            z = jnp.zeros((16,), jnp.float32)
            a0, a1, a2, a3 = lax.fori_loop(0, H, hbody, (z, z, z, z))
            acc_v[j, pl.ds(0, 16)] = a0 * scale
            acc_v[j, pl.ds(16, 16)] = a1 * scale
            acc_v[j, pl.ds(32, 16)] = a2 * scale
            acc_v[j, pl.ds(48, 16)] = a3 * scale
            return carry

        lax.fori_loop(0, BPW, body, 0)
        pltpu.sync_copy(acc_v, out_hbm.at[pl.ds(base, BPW)])

    return pool


def _make_proj(V, D, B, BLK):
    def proj(x_ref, w_ref, b_ref, o_ref):
        acc = lax.dot_general(
            x_ref[...], w_ref[...], (((1,), (1,)), ((), ())),
            preferred_element_type=jnp.float32,
        )
        o_ref[...] = acc + b_ref[...]

    return pl.pallas_call(
        proj,
        grid=(pl.cdiv(V, BLK),),
        in_specs=[
            pl.BlockSpec((B, D), lambda i: (0, 0)),
            pl.BlockSpec((BLK, D), lambda i: (i, 0)),
            pl.BlockSpec((1, BLK), lambda i: (0, i)),
        ],
        out_specs=pl.BlockSpec((B, BLK), lambda i: (0, i)),
        out_shape=jax.ShapeDtypeStruct((B, V), jnp.float32),
    )


def kernel(context, emb_table, W, b):
    H, B = context.shape
    V, D = emb_table.shape
    ctx_bh = context.T.astype(jnp.int32)
    pooled = _make_pool(V, D, B, H)(ctx_bh, emb_table)
    return _make_proj(V, D, B, 512)(pooled, W, b.reshape(1, V))


# speedup vs baseline: 1.1690x; 1.1690x over previous
import functools

import jax
import jax.numpy as jnp
from jax import lax
from jax.experimental import pallas as pl
from jax.experimental.pallas import tpu as pltpu
from jax.experimental.pallas import tpu_sc as plsc


def _make_pool(V, D, B, H):
    info = plsc.get_sparse_core_info()
    NC, NS = info.num_cores, info.num_subcores
    NW = NC * NS
    assert B % NW == 0 and D == 64 and H % 2 == 0
    BPW = B // NW
    mesh = plsc.VectorSubcoreMesh(core_axis_name="c", subcore_axis_name="s")

    @functools.partial(
        pl.kernel,
        mesh=mesh,
        compiler_params=pltpu.CompilerParams(use_tc_tiling_on_sc=False),
        out_type=jax.ShapeDtypeStruct((B, D), jnp.float32),
        scratch_types=[
            pltpu.VMEM((BPW, H), jnp.int32),
            pltpu.VMEM((H, D), jnp.float32),
            pltpu.VMEM((BPW, D), jnp.float32),
            pltpu.SemaphoreType.DMA,
        ],
    )
    def pool(ctx_hbm, table_hbm, out_hbm, idx_v, rows_v, acc_v, sem):
        wid = lax.axis_index("s") * NC + lax.axis_index("c")
        base = wid * BPW
        pltpu.sync_copy(ctx_hbm.at[pl.ds(base, BPW)], idx_v)
        scale = jnp.float32(1.0 / H)

        def body(j, carry):
            pltpu.async_copy(table_hbm.at[idx_v.at[j]], rows_v, sem).wait()

            def hbody(h, acc):
                a0, a1, a2, a3 = acc
                return (
                    a0 + rows_v[h, pl.ds(0, 16)],
                    a1 + rows_v[h, pl.ds(16, 16)],
                    a2 + rows_v[h, pl.ds(32, 16)],
                    a3 + rows_v[h, pl.ds(48, 16)],
                )

            z = jnp.zeros((16,), jnp.float32)
            a0, a1, a2, a3 = lax.fori_loop(0, H, hbody, (z, z, z, z))
            acc_v[j, pl.ds(0, 16)] = a0 * scale
            acc_v[j, pl.ds(16, 16)] = a1 * scale
            acc_v[j, pl.ds(32, 16)] = a2 * scale
            acc_v[j, pl.ds(48, 16)] = a3 * scale
            return carry

        lax.fori_loop(0, BPW, body, 0)
        pltpu.sync_copy(acc_v, out_hbm.at[pl.ds(base, BPW)])

    return pool


def _make_proj(V, D, B, RB):
    def proj(x_ref, wt_ref, b_ref, o_ref):
        acc = lax.dot_general(
            x_ref[...], wt_ref[...], (((1,), (0,)), ((), ())),
            preferred_element_type=jnp.float32,
        )
        o_ref[...] = acc + b_ref[...]

    return pl.pallas_call(
        proj,
        grid=(B // RB,),
        in_specs=[
            pl.BlockSpec((RB, D), lambda i: (i, 0)),
            pl.BlockSpec((D, V), lambda i: (0, 0)),
            pl.BlockSpec((1, V), lambda i: (0, 0)),
        ],
        out_specs=pl.BlockSpec((RB, V), lambda i: (i, 0)),
        out_shape=jax.ShapeDtypeStruct((B, V), jnp.float32),
    )


def kernel(context, emb_table, W, b):
    H, B = context.shape
    V, D = emb_table.shape
    ctx_bh = context.T.astype(jnp.int32)
    pooled = _make_pool(V, D, B, H)(ctx_bh, emb_table)
    return _make_proj(V, D, B, 32)(pooled, W.T, b.reshape(1, V))
